# Initial kernel scaffold; baseline (speedup 1.0000x reference)
import jax
import jax.numpy as jnp
from jax.experimental import pallas as pl


def kernel(dr_vec, Z, neighbor_idxs, embed_table, basis_centers, basis_widths, W_proj):
    raise NotImplementedError("write your pallas kernel here")


# speedup vs baseline: 34.5945x; 34.5945x over previous
import functools

import jax
import jax.numpy as jnp
from jax import lax
from jax.experimental import pallas as pl
from jax.experimental.pallas import tpu as pltpu
from jax.experimental.pallas import tpu_sc as plsc

N = 10000
E = 640000
F = 32
FM = 4 * F
NC, NS, LANES = 2, 16, 16
NW = NC * NS
EPW = E // NW
KBLK = 80
NBLK = EPW // KBLK
NPAD = 10240
RPS = NPAD // NS
ZROWS = 128

EB = 2000
NB = 2048
BN = 640


def _coeff_body(d_ref, ii_ref, jj_ref, c_ref, w_ref, out_ref):
    d = d_ref[...]
    dx = d[:, 0:1]
    dy = d[:, 1:2]
    dz = d[:, 2:3]
    r = jnp.sqrt(dx * dx + dy * dy + dz * dz)
    inv = 1.0 / (r + 1e-5)
    mask = (ii_ref[...] != jj_ref[...]).astype(jnp.float32)
    t = r - c_ref[...]
    basis = jnp.exp(-w_ref[...] * t * t) * mask
    out_ref[...] = jnp.concatenate(
        [basis, basis * (dx * inv), basis * (dy * inv), basis * (dz * inv)],
        axis=1,
    )


def _edge_coeffs(dr_vec, idx_i, idx_j, centers, widths):
    grid = E // EB
    return pl.pallas_call(
        _coeff_body,
        grid=(grid,),
        in_specs=[
            pl.BlockSpec((EB, 3), lambda i: (i, 0)),
            pl.BlockSpec((EB, 1), lambda i: (i, 0)),
            pl.BlockSpec((EB, 1), lambda i: (i, 0)),
            pl.BlockSpec((1, F), lambda i: (0, 0)),
            pl.BlockSpec((1, F), lambda i: (0, 0)),
        ],
        out_specs=pl.BlockSpec((EB, FM), lambda i: (i, 0)),
        out_shape=jax.ShapeDtypeStruct((E, FM), jnp.float32),
    )(dr_vec, idx_i.reshape(E, 1), idx_j.reshape(E, 1),
      centers.reshape(1, F), widths.reshape(1, F))


def _embed_body(z_ref, emb_ref, out_ref):
    z = z_ref[...]
    ids = lax.broadcasted_iota(jnp.int32, (NB, 128), 1)
    onehot = (ids == z).astype(jnp.float32)
    out_ref[...] = jnp.dot(onehot, emb_ref[...],
                           preferred_element_type=jnp.float32)


def _embed(Z, embed_table):
    emb_pad = jnp.zeros((128, 128), jnp.float32).at[:100, :F].set(embed_table)
    z_pad = jnp.zeros((NPAD,), jnp.int32).at[:N].set(Z)
    return pl.pallas_call(
        _embed_body,
        grid=(NPAD // NB,),
        in_specs=[
            pl.BlockSpec((NB, 1), lambda i: (i, 0)),
            pl.BlockSpec((128, 128), lambda i: (0, 0)),
        ],
        out_specs=pl.BlockSpec((NB, 128), lambda i: (i, 0)),
        out_shape=jax.ShapeDtypeStruct((NPAD, 128), jnp.float32),
    )(z_pad.reshape(NPAD, 1), emb_pad)


def _sc_moments_body(h_hbm, b4_hbm, ii_hbm, jj_hbm, out_hbm,
                     idxi_v, idxj_v, hrows_v, b4_v, v_v, zbuf_v, acc_sh,
                     sem):
    cid = lax.axis_index("c")
    sid = lax.axis_index("s")
    wid = sid * NC + cid

    def zrow(i, carry):
        for c in range(FM // LANES):
            zbuf_v[i, pl.ds(c * LANES, LANES)] = jnp.zeros((LANES,),
                                                           jnp.float32)
        return carry
    lax.fori_loop(0, ZROWS, zrow, 0)

    def zcopy(j, carry):
        pltpu.sync_copy(zbuf_v,
                        acc_sh.at[pl.ds(sid * RPS + j * ZROWS, ZROWS)])
        return carry
    lax.fori_loop(0, RPS // ZROWS, zcopy, 0)
    plsc.subcore_barrier()

    base = wid * EPW

    def blk(b, carry):
        e0 = base + b * KBLK
        pltpu.sync_copy(ii_hbm.at[pl.ds(e0, KBLK)], idxi_v)
        pltpu.sync_copy(jj_hbm.at[pl.ds(e0, KBLK)], idxj_v)
        pltpu.async_copy(h_hbm.at[idxj_v], hrows_v, sem).wait()
        pltpu.sync_copy(b4_hbm.at[pl.ds(e0, KBLK)], b4_v)

        def edge(k, inner):
            h0 = hrows_v[k, pl.ds(0, LANES)]
            h1 = hrows_v[k, pl.ds(LANES, LANES)]
            for c in range(FM // LANES):
                hh = h0 if (c % 2 == 0) else h1
                v_v[k, pl.ds(c * LANES, LANES)] = (
                    hh * b4_v[k, pl.ds(c * LANES, LANES)])
            return inner
        lax.fori_loop(0, KBLK, edge, 0)

        pltpu.sync_copy(v_v, acc_sh.at[idxi_v], add=True)
        return carry
    lax.fori_loop(0, NBLK, blk, 0)

    plsc.subcore_barrier()
    pltpu.sync_copy(acc_sh.at[pl.ds(sid * RPS, RPS)],
                    out_hbm.at[cid, pl.ds(sid * RPS, RPS)])


def _sc_moments(h, b4, idx_i, idx_j):
    mesh = plsc.VectorSubcoreMesh(core_axis_name="c", subcore_axis_name="s",
                                  num_cores=NC, num_subcores=NS)
    f = pl.kernel(
        _sc_moments_body,
        out_type=jax.ShapeDtypeStruct((NC, NPAD, FM), jnp.float32),
        mesh=mesh,
        scratch_types=[
            pltpu.VMEM((KBLK,), jnp.int32),
            pltpu.VMEM((KBLK,), jnp.int32),
            pltpu.VMEM((KBLK, 128), jnp.float32),
            pltpu.VMEM((KBLK, FM), jnp.float32),
            pltpu.VMEM((KBLK, FM), jnp.float32),
            pltpu.VMEM((ZROWS, FM), jnp.float32),
            pltpu.VMEM_SHARED((NPAD, FM), jnp.float32),
            pltpu.SemaphoreType.DMA,
        ],
    )
    return f(h, b4, idx_i, idx_j)


def _post_proj_body(m_ref, wp_ref, out_ref):
    m = m_ref[0] + m_ref[1]
    m0 = m[:, 0:F]
    mx = m[:, F:2 * F]
    my = m[:, 2 * F:3 * F]
    mz = m[:, 3 * F:4 * F]
    acc = jnp.dot(m0, wp_ref[0:F, :], preferred_element_type=jnp.float32)
    for r in range(F):
        cr = (mx[:, r:r + 1] * mx + my[:, r:r + 1] * my
              + mz[:, r:r + 1] * mz)
        acc = acc + jnp.dot(cr, wp_ref[F + r * F:F + (r + 1) * F, :],
                            preferred_element_type=jnp.float32)
    out_ref[...] = jnp.concatenate(
        [acc, jnp.zeros((BN, 128 - F), jnp.float32)], axis=1)


def _post_proj(moments, w_proj):
    return pl.pallas_call(
        _post_proj_body,
        grid=(NPAD // BN,),
        in_specs=[
            pl.BlockSpec((NC, BN, FM), lambda i: (0, i, 0)),
            pl.BlockSpec((F + F * F, F), lambda i: (0, 0)),
        ],
        out_specs=pl.BlockSpec((BN, 128), lambda i: (i, 0)),
        out_shape=jax.ShapeDtypeStruct((NPAD, 128), jnp.float32),
    )(moments, w_proj)


def _post_out_body(m_ref, out_ref):
    m = m_ref[0] + m_ref[1]
    m0 = m[:, 0:F]
    mx = m[:, F:2 * F]
    my = m[:, 2 * F:3 * F]
    mz = m[:, 3 * F:4 * F]
    out_ref[:, 0:F] = m0
    for r in range(F):
        cr = (mx[:, r:r + 1] * mx + my[:, r:r + 1] * my
              + mz[:, r:r + 1] * mz)
        out_ref[:, F + r * F:F + (r + 1) * F] = cr


def _post_out(moments):
    return pl.pallas_call(
        _post_out_body,
        grid=(NPAD // BN,),
        in_specs=[pl.BlockSpec((NC, BN, FM), lambda i: (0, i, 0))],
        out_specs=pl.BlockSpec((BN, F + F * F), lambda i: (i, 0)),
        out_shape=jax.ShapeDtypeStruct((NPAD, F + F * F), jnp.float32),
    )(moments)


def kernel(dr_vec, Z, neighbor_idxs, embed_table, basis_centers,
           basis_widths, W_proj):
    dr_vec = dr_vec.astype(jnp.float32)
    idx_i = neighbor_idxs[0]
    idx_j = neighbor_idxs[1]

    b4 = _edge_coeffs(dr_vec, idx_i, idx_j, basis_centers, basis_widths)
    h = _embed(Z, embed_table)

    m1 = _sc_moments(h, b4, idx_i, idx_j)
    h1 = _post_proj(m1, W_proj)
    m2 = _sc_moments(h1, b4, idx_i, idx_j)
    return _post_out(m2)[:N]
